# baseline (speedup 1.0000x reference)
import functools

import jax
import jax.numpy as jnp
from jax import lax
from jax.experimental import pallas as pl
from jax.experimental.pallas import tpu as pltpu
from jax.experimental.pallas import tpu_sc as plsc

B, C, HW, P = 16, 3, 224, 16
GH = HW // P
K, D = 1024, 256
PD = C * P * P
N = B * GH * GH

TM = 392
NT = N // TM

NC, NS = 2, 16
NW = NC * NS
BPW = 104
BP = NW * BPW


def _vq_body(z_ref, zsq_ref, cb_ref, idx_ref):
    cbv = cb_ref[...]
    e_sq = jnp.sum(cbv * cbv, axis=1, keepdims=True)
    zc = lax.dot_general(z_ref[...], cbv, (((1,), (1,)), ((), ())),
                         precision=lax.Precision.DEFAULT,
                         preferred_element_type=jnp.float32)
    dist = (zsq_ref[:, 0:1] + e_sq.T) - 2.0 * zc
    minv = jnp.min(dist, axis=1, keepdims=True)
    iota = lax.broadcasted_iota(jnp.int32, (TM, K), 1)
    idx_ref[0, 0, :] = jnp.min(jnp.where(dist == minv, iota, K), axis=1)


def _vq(z, zsq_bc, codebook):
    return pl.pallas_call(
        _vq_body,
        grid=(NT,),
        in_specs=[
            pl.BlockSpec((TM, D), lambda i: (i, 0)),
            pl.BlockSpec((TM, 128), lambda i: (i, 0)),
            pl.BlockSpec((K, D), lambda i: (0, 0)),
        ],
        out_specs=pl.BlockSpec((1, 1, TM), lambda i: (i, 0, 0)),
        out_shape=jax.ShapeDtypeStruct((NT, 1, TM), jnp.int32),
    )(z, zsq_bc, codebook)


def _table_body(cb_ref, wd_ref, bd_ref, tab_ref):
    tab_ref[...] = jnp.dot(cb_ref[...], wd_ref[...],
                           preferred_element_type=jnp.float32) + bd_ref[...]


def _dec_table(codebook, W_dec, b_dec):
    return pl.pallas_call(
        _table_body,
        out_shape=jax.ShapeDtypeStruct((K, PD), jnp.float32),
    )(codebook, W_dec, b_dec)


def _sc_gather(table, idx_pad):
    mesh = plsc.VectorSubcoreMesh(core_axis_name="c", subcore_axis_name="s",
                                  num_cores=NC, num_subcores=NS)

    @functools.partial(
        pl.kernel,
        mesh=mesh,
        out_type=jax.ShapeDtypeStruct((BP, PD), jnp.float32),
        scratch_types=[
            pltpu.VMEM((BPW,), jnp.int32),
            pltpu.VMEM((BPW, PD), jnp.float32),
            pltpu.SemaphoreType.DMA,
        ],
    )
    def gather(table_hbm, idx_hbm, out_hbm, idx_v, rows_v, sem):
        wid = lax.axis_index("s") * NC + lax.axis_index("c")
        base = wid * BPW
        pltpu.sync_copy(idx_hbm.at[pl.ds(base, BPW)], idx_v)
        pltpu.async_copy(table_hbm.at[idx_v], rows_v, sem).wait()
        pltpu.sync_copy(rows_v, out_hbm.at[pl.ds(base, BPW)])

    return gather(table, idx_pad)


def kernel(x, codebook, W_enc, b_enc, W_dec, b_dec):
    xp = (x.reshape(B, C, GH, P, GH, P)
           .transpose(0, 2, 4, 1, 3, 5)
           .reshape(N, PD))
    z_e = xp @ W_enc + b_enc
    z_flat = z_e.reshape(N, D)
    z_sq = jnp.sum(z_flat ** 2, axis=1, keepdims=True)
    zsq_bc = jnp.broadcast_to(z_sq, (N, 128))
    idx = _vq(z_flat, zsq_bc, codebook).reshape(N)
    table = _dec_table(codebook, W_dec, b_dec)
    idx_pad = jnp.concatenate([idx, jnp.zeros((BP - N,), jnp.int32)])
    rows = _sc_gather(table, idx_pad)
    x_recon = (rows[:N].reshape(B, GH, GH, C, P, P)
               .transpose(0, 3, 1, 4, 2, 5)
               .reshape(B, C, HW, HW))
    codes = idx.reshape(B, GH, GH)
    return x_recon, codes


# speedup vs baseline: 1.0715x; 1.0715x over previous
import functools

import jax
import jax.numpy as jnp
from jax import lax
from jax.experimental import pallas as pl
from jax.experimental.pallas import tpu as pltpu
from jax.experimental.pallas import tpu_sc as plsc

B, C, HW, P = 16, 3, 224, 16
GH = HW // P
K, D = 1024, 256
PD = C * P * P
N = B * GH * GH

TM = 392
NT = N // TM

NC, NS = 2, 16
NW = NC * NS
BPW = 104


def _vq_body(z_ref, zsq_ref, cb_ref, wd_ref, bd_ref, idx_ref, tab_ref):
    @pl.when(pl.program_id(0) == 0)
    def _():
        tab_ref[...] = jnp.dot(cb_ref[...], wd_ref[...],
                               preferred_element_type=jnp.float32) + bd_ref[...]

    cbv = cb_ref[...]
    e_sq = jnp.sum(cbv * cbv, axis=1, keepdims=True)
    zc = lax.dot_general(z_ref[...], cbv, (((1,), (1,)), ((), ())),
                         precision=lax.Precision.DEFAULT,
                         preferred_element_type=jnp.float32)
    dist = (zsq_ref[...] + e_sq.T) - 2.0 * zc
    minv = jnp.min(dist, axis=1, keepdims=True)
    iota = lax.broadcasted_iota(jnp.int32, (TM, K), 1)
    idx_ref[0, 0, :] = jnp.min(jnp.where(dist == minv, iota, K), axis=1)


def _vq_and_table(z, z_sq, codebook, W_dec, b_dec):
    return pl.pallas_call(
        _vq_body,
        grid=(NT,),
        in_specs=[
            pl.BlockSpec((TM, D), lambda i: (i, 0)),
            pl.BlockSpec((TM, 1), lambda i: (i, 0)),
            pl.BlockSpec((K, D), lambda i: (0, 0)),
            pl.BlockSpec((D, PD), lambda i: (0, 0)),
            pl.BlockSpec((PD,), lambda i: (0,)),
        ],
        out_specs=[
            pl.BlockSpec((1, 1, TM), lambda i: (i, 0, 0)),
            pl.BlockSpec((K, PD), lambda i: (0, 0)),
        ],
        out_shape=[
            jax.ShapeDtypeStruct((NT, 1, TM), jnp.int32),
            jax.ShapeDtypeStruct((K, PD), jnp.float32),
        ],
    )(z, z_sq, codebook, W_dec, b_dec)


def _sc_gather(table, idx):
    mesh = plsc.VectorSubcoreMesh(core_axis_name="c", subcore_axis_name="s",
                                  num_cores=NC, num_subcores=NS)

    @functools.partial(
        pl.kernel,
        mesh=mesh,
        out_type=jax.ShapeDtypeStruct((N, PD), jnp.float32),
        scratch_types=[
            pltpu.VMEM((BPW,), jnp.int32),
            pltpu.VMEM((BPW, PD), jnp.float32),
            pltpu.SemaphoreType.DMA,
        ],
    )
    def gather(table_hbm, idx_hbm, out_hbm, idx_v, rows_v, sem):
        wid = lax.axis_index("s") * NC + lax.axis_index("c")
        base = jnp.minimum(wid * BPW, N - BPW)
        pltpu.sync_copy(idx_hbm.at[pl.ds(base, BPW)], idx_v)
        pltpu.async_copy(table_hbm.at[idx_v], rows_v, sem).wait()
        pltpu.sync_copy(rows_v, out_hbm.at[pl.ds(base, BPW)])

    return gather(table, idx)


def kernel(x, codebook, W_enc, b_enc, W_dec, b_dec):
    xp = (x.reshape(B, C, GH, P, GH, P)
           .transpose(0, 2, 4, 1, 3, 5)
           .reshape(N, PD))
    z_e = xp @ W_enc + b_enc
    z_flat = z_e.reshape(N, D)
    z_sq = jnp.sum(z_flat ** 2, axis=1, keepdims=True)
    idx3, table = _vq_and_table(z_flat, z_sq, codebook, W_dec, b_dec)
    idx = idx3.reshape(N)
    rows = _sc_gather(table, idx)
    x_recon = (rows.reshape(B, GH, GH, C, P, P)
               .transpose(0, 3, 1, 4, 2, 5)
               .reshape(B, C, HW, HW))
    codes = idx.reshape(B, GH, GH)
    return x_recon, codes


# speedup vs baseline: 1.6288x; 1.5201x over previous
import functools

import jax
import jax.numpy as jnp
from jax import lax
from jax.experimental import pallas as pl
from jax.experimental.pallas import tpu as pltpu
from jax.experimental.pallas import tpu_sc as plsc

B, C, HW, P = 16, 3, 224, 16
GH = HW // P
K, D = 1024, 256
PD = C * P * P
N = B * GH * GH

TM = 392
NT = N // TM

NC, NS = 2, 16
NW = NC * NS

NCHUNK = PD // P
NROW = N * NCHUNK
RPW = NROW // NW
CH = 96
NDMA = RPW // CH


def _vq_body(z_ref, zsq_ref, cb_ref, wd_ref, bd_ref, idx_ref, tab_ref):
    @pl.when(pl.program_id(0) == 0)
    def _():
        tab_ref[...] = jnp.dot(cb_ref[...], wd_ref[...],
                               preferred_element_type=jnp.float32) + bd_ref[...]

    cbv = cb_ref[...]
    e_sq = jnp.sum(cbv * cbv, axis=1, keepdims=True)
    zc = lax.dot_general(z_ref[...], cbv, (((1,), (1,)), ((), ())),
                         precision=lax.Precision.DEFAULT,
                         preferred_element_type=jnp.float32)
    dist = (zsq_ref[...] + e_sq.T) - 2.0 * zc
    minv = jnp.min(dist, axis=1, keepdims=True)
    iota = lax.broadcasted_iota(jnp.int32, (TM, K), 1)
    idx_ref[0, 0, :] = jnp.min(jnp.where(dist == minv, iota, K), axis=1)


def _vq_and_table(z, z_sq, codebook, W_dec, b_dec):
    return pl.pallas_call(
        _vq_body,
        grid=(NT,),
        in_specs=[
            pl.BlockSpec((TM, D), lambda i: (i, 0)),
            pl.BlockSpec((TM, 1), lambda i: (i, 0)),
            pl.BlockSpec((K, D), lambda i: (0, 0)),
            pl.BlockSpec((D, PD), lambda i: (0, 0)),
            pl.BlockSpec((PD,), lambda i: (0,)),
        ],
        out_specs=[
            pl.BlockSpec((1, 1, TM), lambda i: (i, 0, 0)),
            pl.BlockSpec((K, PD), lambda i: (0, 0)),
        ],
        out_shape=[
            jax.ShapeDtypeStruct((NT, 1, TM), jnp.int32),
            jax.ShapeDtypeStruct((K, PD), jnp.float32),
        ],
    )(z, z_sq, codebook, W_dec, b_dec)


def _sc_gather(table2, src):
    mesh = plsc.VectorSubcoreMesh(core_axis_name="c", subcore_axis_name="s",
                                  num_cores=NC, num_subcores=NS)

    @functools.partial(
        pl.kernel,
        mesh=mesh,
        out_type=jax.ShapeDtypeStruct((NROW, P), jnp.float32),
        compiler_params=pltpu.CompilerParams(use_tc_tiling_on_sc=False),
        scratch_types=[
            pltpu.VMEM((RPW,), jnp.int32),
            pltpu.VMEM((RPW, P), jnp.float32),
            pltpu.SemaphoreType.DMA,
        ],
    )
    def gather(tab_hbm, src_hbm, out_hbm, idx_v, slab_v, sem):
        wid = lax.axis_index("s") * NC + lax.axis_index("c")
        base = wid * RPW
        pltpu.sync_copy(src_hbm.at[pl.ds(base, RPW)], idx_v)

        def body(j, carry):
            o = j * CH
            pltpu.async_copy(tab_hbm.at[idx_v.at[pl.ds(o, CH)]],
                             slab_v.at[pl.ds(o, CH)], sem)
            return carry

        lax.fori_loop(0, NDMA, body, 0)
        pltpu.make_async_copy(tab_hbm.at[pl.ds(0, RPW)], slab_v, sem).wait()
        pltpu.sync_copy(slab_v, out_hbm.at[pl.ds(base, RPW)])

    return gather(table2, src)


def kernel(x, codebook, W_enc, b_enc, W_dec, b_dec):
    xp = (x.reshape(B, C, GH, P, GH, P)
           .transpose(0, 2, 4, 1, 3, 5)
           .reshape(N, PD))
    z_e = xp @ W_enc + b_enc
    z_flat = z_e.reshape(N, D)
    z_sq = jnp.sum(z_flat ** 2, axis=1, keepdims=True)
    idx3, table = _vq_and_table(z_flat, z_sq, codebook, W_dec, b_dec)
    idx = idx3.reshape(N)
    table2 = table.reshape(K, NCHUNK, P).transpose(1, 0, 2).reshape(NCHUNK * K, P)
    offs = (jnp.arange(NCHUNK, dtype=jnp.int32) * K)[:, None]
    src = (offs + idx[None, :]).reshape(C, P, B, GH, GH)
    src = src.transpose(2, 0, 3, 1, 4).reshape(NROW)
    x_recon = _sc_gather(table2, src).reshape(B, C, HW, HW)
    codes = idx.reshape(B, GH, GH)
    return x_recon, codes
